# baseline (speedup 1.0000x reference)
import functools
import jax
import jax.numpy as jnp
from jax import lax
from jax.experimental import pallas as pl
from jax.experimental.pallas import tpu as pltpu

B, N, DIM = 1, 2048, 1024
HEADS = 16
HD = DIM // HEADS
E, S, TOPK = 8, 2, 2
HID = 512

TB = 256
NBLK = N // TB


def _ln(x, g, b, eps=1e-5):
    m = x.mean(-1, keepdims=True)
    v = ((x - m) ** 2).mean(-1, keepdims=True)
    return (x - m) * lax.rsqrt(v + eps) * g + b


def _mm(x, w):
    return lax.dot_general(x, w, (((1,), (1,)), ((), ())),
                           preferred_element_type=jnp.float32)


GP = 8
GR = GP * HEADS


def _attn_block(hl, wqkv, bqkv, wo, bo):
    qkv = _mm(hl, wqkv) + bqkv
    q = qkv[:, :DIM].reshape(TB, HEADS, HD)
    k = qkv[:, DIM:2 * DIM].reshape(TB, HEADS, HD)
    v = qkv[:, 2 * DIM:].reshape(TB, HEADS, HD)
    scale = 1.0 / (HD ** 0.5)
    s = lax.dot_general(q, k, (((2,), (2,)), ((0,), (0,))),
                        preferred_element_type=jnp.float32) * scale
    s = s - jnp.max(s, axis=-1, keepdims=True)
    p = jnp.exp(s)
    p = p / jnp.sum(p, axis=-1, keepdims=True)
    out = lax.dot_general(p, v, (((2,), (1,)), ((0,), (0,))),
                          preferred_element_type=jnp.float32)
    out = out.reshape(TB, DIM)
    return _mm(out, wo) + bo


def _k1_body(h_ref, e_ref, loga_ref, ltiw_ref, ltib_ref, wqkv_ref, bqkv_ref,
             wo_ref, bo_ref, n1g_ref, n1b_ref, n2g_ref, n2b_ref, h2_ref):
    a_vec = -jnp.exp(loga_ref[...])
    edge_inj = _mm(e_ref[...], ltiw_ref[...]) + ltib_ref[...]
    hl = a_vec * h_ref[...] + edge_inj
    hl = _ln(hl, n1g_ref[...], n1b_ref[...])
    h_attn = _attn_block(hl, wqkv_ref[...], bqkv_ref[...], wo_ref[...],
                         bo_ref[...])
    h2_ref[...] = _ln(hl + h_attn, n2g_ref[...], n2b_ref[...])


def _k2_body(h2_ref, loga_ref, rw_ref, ew1_ref, eb1_ref, ew2_ref, eb2_ref,
             sw1_ref, sb1_ref, sw2_ref, sb2_ref, hw1_ref, hb1_ref, hw2_ref,
             hb2_ref, n3g_ref, n3b_ref,
             h3_ref, halt_ref, counts_ref, sprob_ref, loss_ref):
    pid = pl.program_id(0)
    x = h2_ref[...]
    logits = jnp.dot(x, rw_ref[...], preferred_element_type=jnp.float32)
    logits = logits - jnp.max(logits, axis=-1, keepdims=True)
    pexp = jnp.exp(logits)
    probs = pexp / jnp.sum(pexp, axis=-1, keepdims=True)

    lanes = lax.broadcasted_iota(jnp.int32, (TB, E), 1)
    i0 = jnp.argmax(probs, axis=-1)[:, None]
    v0 = jnp.max(probs, axis=-1, keepdims=True)
    masked = jnp.where(lanes == i0, -jnp.inf, probs)
    i1 = jnp.argmax(masked, axis=-1)[:, None]
    v1 = jnp.max(masked, axis=-1, keepdims=True)
    denom = v0 + v1
    sel0 = (lanes == i0).astype(jnp.float32)
    sel1 = (lanes == i1).astype(jnp.float32)
    coef = (v0 / denom) * sel0 + (v1 / denom) * sel1

    xb = x.astype(jnp.bfloat16)
    routed = jnp.zeros((TB, DIM), jnp.float32)
    for e in range(E):
        eh = jax.nn.silu(jnp.dot(xb, ew1_ref[e],
                                 preferred_element_type=jnp.float32)
                         + eb1_ref[e])
        eo = jnp.dot(eh.astype(jnp.bfloat16), ew2_ref[e],
                     preferred_element_type=jnp.float32) + eb2_ref[e]
        routed = routed + coef[:, e:e + 1] * eo
    for s in range(S):
        sh = jax.nn.silu(jnp.dot(xb, sw1_ref[s],
                                 preferred_element_type=jnp.float32)
                         + sb1_ref[s])
        routed = routed + jnp.dot(sh.astype(jnp.bfloat16), sw2_ref[s],
                                  preferred_element_type=jnp.float32) \
            + sb2_ref[s]

    h3 = _ln(x + routed, n3g_ref[...], n3b_ref[...])
    h3_ref[...] = h3

    t1 = jnp.tanh(lax.dot_general(h3.astype(jnp.bfloat16), hw1_ref[...],
                                  (((1,), (1,)), ((), ())),
                                  preferred_element_type=jnp.float32)
                  + hb1_ref[...])
    hl = jnp.sum(t1 * hw2_ref[...], axis=-1, keepdims=True) + hb2_ref[0, 0]
    halt_ref[...] = jax.nn.sigmoid(hl)

    @pl.when(pid == 0)
    def _init():
        counts_ref[...] = jnp.zeros_like(counts_ref)
        sprob_ref[...] = jnp.zeros_like(sprob_ref)

    counts_ref[...] += jnp.sum(sel0 + sel1, axis=0, keepdims=True)
    sprob_ref[...] += jnp.sum(probs, axis=0, keepdims=True)

    @pl.when(pid == NBLK - 1)
    def _fin():
        radius = jnp.max(jnp.abs(jnp.exp(loga_ref[...])))
        stab = jnp.maximum(radius - 0.95, 0.0)
        t = jnp.float32(N)
        moe = E * jnp.sum(counts_ref[...] * sprob_ref[...]) / (t * t)
        loss_ref[...] = jnp.full((1, 1), stab + moe, jnp.float32)


def _row_spec(i_map=None):
    return pl.BlockSpec((TB, DIM), lambda i: (i, 0))


def _full(shape):
    nd = len(shape)
    return pl.BlockSpec(shape, lambda i: (0,) * nd)


@jax.jit
def kernel(h, edge_features, log_A, lti_B_w, lti_B_b, Wq, bq, Wk, bk, Wv, bv,
           Wo, bo, n1_g, n1_b, n2_g, n2_b, n3_g, n3_b, router_w, ew1, eb1,
           ew2, eb2, sw1, sb1, sw2, sb2, hw1, hb1, hw2, hb2):
    hf = h.reshape(N, DIM)
    ef = edge_features.reshape(N, DIM)
    wqkv = jnp.concatenate([Wq, Wk, Wv], axis=0)
    bqkv = jnp.concatenate([bq, bk, bv], axis=0).reshape(1, 3 * DIM)
    r1 = lambda a: a.reshape(1, -1)

    h2 = pl.pallas_call(
        _k1_body,
        grid=(NBLK,),
        in_specs=[
            _row_spec(), _row_spec(), _full((1, DIM)), _full((DIM, DIM)),
            _full((1, DIM)), _full((3 * DIM, DIM)), _full((1, 3 * DIM)),
            _full((DIM, DIM)), _full((1, DIM)), _full((1, DIM)),
            _full((1, DIM)), _full((1, DIM)), _full((1, DIM)),
        ],
        out_specs=_row_spec(),
        out_shape=jax.ShapeDtypeStruct((N, DIM), jnp.float32),
    )(hf, ef, r1(log_A), lti_B_w, r1(lti_B_b), wqkv, bqkv, Wo, r1(bo),
      r1(n1_g), r1(n1_b), r1(n2_g), r1(n2_b))

    h3, halt, _, _, loss = pl.pallas_call(
        _k2_body,
        grid=(NBLK,),
        in_specs=[
            _row_spec(), _full((1, DIM)), _full((DIM, E)),
            _full((E, DIM, HID)), _full((E, 1, HID)),
            _full((E, HID, DIM)), _full((E, 1, DIM)),
            _full((S, DIM, HID)), _full((S, 1, HID)),
            _full((S, HID, DIM)), _full((S, 1, DIM)),
            _full((DIM // 2, DIM)), _full((1, DIM // 2)),
            _full((1, DIM // 2)), _full((1, 1)),
            _full((1, DIM)), _full((1, DIM)),
        ],
        out_specs=[
            _row_spec(),
            pl.BlockSpec((TB, 1), lambda i: (i, 0)),
            _full((1, E)), _full((1, E)), _full((1, 1)),
        ],
        out_shape=[
            jax.ShapeDtypeStruct((N, DIM), jnp.float32),
            jax.ShapeDtypeStruct((N, 1), jnp.float32),
            jax.ShapeDtypeStruct((1, E), jnp.float32),
            jax.ShapeDtypeStruct((1, E), jnp.float32),
            jax.ShapeDtypeStruct((1, 1), jnp.float32),
        ],
    )(h2, r1(log_A), router_w, ew1.astype(jnp.bfloat16),
      eb1.reshape(E, 1, HID), ew2.astype(jnp.bfloat16),
      eb2.reshape(E, 1, DIM), sw1.astype(jnp.bfloat16),
      sb1.reshape(S, 1, HID), sw2.astype(jnp.bfloat16),
      sb2.reshape(S, 1, DIM), hw1.astype(jnp.bfloat16), r1(hb1), hw2,
      hb2.reshape(1, 1), r1(n3_g), r1(n3_b))

    h3 = h3.reshape(B, N, DIM)
    halt = halt.reshape(B, N)
    loss = loss.reshape(())
    return h3, halt, halt, loss


# speedup vs baseline: 1.1213x; 1.1213x over previous
import functools
import jax
import jax.numpy as jnp
from jax import lax
from jax.experimental import pallas as pl
from jax.experimental.pallas import tpu as pltpu

B, N, DIM = 1, 2048, 1024
HEADS = 16
HD = DIM // HEADS
E, S, TOPK = 8, 2, 2
HID = 512

TB = 256
NBLK = N // TB


def _ln(x, g, b, eps=1e-5):
    m = x.mean(-1, keepdims=True)
    v = ((x - m) ** 2).mean(-1, keepdims=True)
    return (x - m) * lax.rsqrt(v + eps) * g + b


def _mm(x, w):
    return lax.dot_general(x, w, (((1,), (1,)), ((), ())),
                           preferred_element_type=jnp.float32)


GP = 8
GR = GP * HEADS


def _attn_block(hl, wqkv, bqkv, wo, bo):
    qkv = _mm(hl, wqkv) + bqkv
    q = qkv[:, :DIM].reshape(TB, HEADS, HD)
    k = qkv[:, DIM:2 * DIM].reshape(TB, HEADS, HD)
    v = qkv[:, 2 * DIM:].reshape(TB, HEADS, HD)
    scale = 1.0 / (HD ** 0.5)
    s = lax.dot_general(q, k, (((2,), (2,)), ((0,), (0,))),
                        preferred_element_type=jnp.float32) * scale
    s = s - jnp.max(s, axis=-1, keepdims=True)
    p = jnp.exp(s)
    p = p / jnp.sum(p, axis=-1, keepdims=True)
    out = lax.dot_general(p, v, (((2,), (1,)), ((0,), (0,))),
                          preferred_element_type=jnp.float32)
    out = out.reshape(TB, DIM)
    return _mm(out, wo) + bo


def _k1_body(h_ref, e_ref, loga_ref, ltiw_ref, ltib_ref, wqkv_ref, bqkv_ref,
             wo_ref, bo_ref, n1g_ref, n1b_ref, n2g_ref, n2b_ref, h2_ref):
    a_vec = -jnp.exp(loga_ref[...])
    edge_inj = _mm(e_ref[...], ltiw_ref[...]) + ltib_ref[...]
    hl = a_vec * h_ref[...] + edge_inj
    hl = _ln(hl, n1g_ref[...], n1b_ref[...])
    h_attn = _attn_block(hl, wqkv_ref[...], bqkv_ref[...], wo_ref[...],
                         bo_ref[...])
    h2_ref[...] = _ln(hl + h_attn, n2g_ref[...], n2b_ref[...])


def _k2_body(h2_ref, loga_ref, rw_ref, ew1_ref, eb1_ref, ew2_ref, eb2_ref,
             sw1_ref, sb1_ref, sw2_ref, sb2_ref, hw1_ref, hb1_ref, hw2_ref,
             hb2_ref, n3g_ref, n3b_ref,
             h3_ref, halt_ref, counts_ref, sprob_ref, loss_ref):
    pid = pl.program_id(0)
    x = h2_ref[...]
    logits = jnp.dot(x, rw_ref[...], preferred_element_type=jnp.float32)
    logits = logits - jnp.max(logits, axis=-1, keepdims=True)
    pexp = jnp.exp(logits)
    probs = pexp / jnp.sum(pexp, axis=-1, keepdims=True)

    lanes = lax.broadcasted_iota(jnp.int32, (TB, E), 1)
    i0 = jnp.argmax(probs, axis=-1)[:, None]
    v0 = jnp.max(probs, axis=-1, keepdims=True)
    masked = jnp.where(lanes == i0, -jnp.inf, probs)
    i1 = jnp.argmax(masked, axis=-1)[:, None]
    v1 = jnp.max(masked, axis=-1, keepdims=True)
    denom = v0 + v1
    sel0 = (lanes == i0).astype(jnp.float32)
    sel1 = (lanes == i1).astype(jnp.float32)
    coef = (v0 / denom) * sel0 + (v1 / denom) * sel1

    routed = jnp.zeros((TB, DIM), jnp.float32)
    for e in range(E):
        eh = jax.nn.silu(jnp.dot(x, ew1_ref[e],
                                 preferred_element_type=jnp.float32)
                         + eb1_ref[e])
        eo = jnp.dot(eh, ew2_ref[e], preferred_element_type=jnp.float32) \
            + eb2_ref[e]
        routed = routed + coef[:, e:e + 1] * eo
    for s in range(S):
        sh = jax.nn.silu(jnp.dot(x, sw1_ref[s],
                                 preferred_element_type=jnp.float32)
                         + sb1_ref[s])
        routed = routed + jnp.dot(sh, sw2_ref[s],
                                  preferred_element_type=jnp.float32) \
            + sb2_ref[s]

    h3 = _ln(x + routed, n3g_ref[...], n3b_ref[...])
    h3_ref[...] = h3

    t1 = jnp.tanh(_mm(h3, hw1_ref[...]) + hb1_ref[...])
    hl = jnp.sum(t1 * hw2_ref[...], axis=-1, keepdims=True) + hb2_ref[0, 0]
    halt_ref[...] = jax.nn.sigmoid(hl)

    @pl.when(pid == 0)
    def _init():
        counts_ref[...] = jnp.zeros_like(counts_ref)
        sprob_ref[...] = jnp.zeros_like(sprob_ref)

    counts_ref[...] += jnp.sum(sel0 + sel1, axis=0, keepdims=True)
    sprob_ref[...] += jnp.sum(probs, axis=0, keepdims=True)

    @pl.when(pid == NBLK - 1)
    def _fin():
        radius = jnp.max(jnp.abs(jnp.exp(loga_ref[...])))
        stab = jnp.maximum(radius - 0.95, 0.0)
        t = jnp.float32(N)
        moe = E * jnp.sum(counts_ref[...] * sprob_ref[...]) / (t * t)
        loss_ref[...] = jnp.full((1, 1), stab + moe, jnp.float32)


def _row_spec(i_map=None):
    return pl.BlockSpec((TB, DIM), lambda i: (i, 0))


def _full(shape):
    nd = len(shape)
    return pl.BlockSpec(shape, lambda i: (0,) * nd)


@jax.jit
def kernel(h, edge_features, log_A, lti_B_w, lti_B_b, Wq, bq, Wk, bk, Wv, bv,
           Wo, bo, n1_g, n1_b, n2_g, n2_b, n3_g, n3_b, router_w, ew1, eb1,
           ew2, eb2, sw1, sb1, sw2, sb2, hw1, hb1, hw2, hb2):
    hf = h.reshape(N, DIM)
    ef = edge_features.reshape(N, DIM)
    wqkv = jnp.concatenate([Wq, Wk, Wv], axis=0)
    bqkv = jnp.concatenate([bq, bk, bv], axis=0).reshape(1, 3 * DIM)
    r1 = lambda a: a.reshape(1, -1)

    h2 = pl.pallas_call(
        _k1_body,
        grid=(NBLK,),
        in_specs=[
            _row_spec(), _row_spec(), _full((1, DIM)), _full((DIM, DIM)),
            _full((1, DIM)), _full((3 * DIM, DIM)), _full((1, 3 * DIM)),
            _full((DIM, DIM)), _full((1, DIM)), _full((1, DIM)),
            _full((1, DIM)), _full((1, DIM)), _full((1, DIM)),
        ],
        out_specs=_row_spec(),
        out_shape=jax.ShapeDtypeStruct((N, DIM), jnp.float32),
    )(hf, ef, r1(log_A), lti_B_w, r1(lti_B_b), wqkv, bqkv, Wo, r1(bo),
      r1(n1_g), r1(n1_b), r1(n2_g), r1(n2_b))

    h3, halt, _, _, loss = pl.pallas_call(
        _k2_body,
        grid=(NBLK,),
        in_specs=[
            _row_spec(), _full((1, DIM)), _full((DIM, E)),
            _full((E, DIM, HID)), _full((E, 1, HID)),
            _full((E, HID, DIM)), _full((E, 1, DIM)),
            _full((S, DIM, HID)), _full((S, 1, HID)),
            _full((S, HID, DIM)), _full((S, 1, DIM)),
            _full((DIM // 2, DIM)), _full((1, DIM // 2)),
            _full((1, DIM // 2)), _full((1, 1)),
            _full((1, DIM)), _full((1, DIM)),
        ],
        out_specs=[
            _row_spec(),
            pl.BlockSpec((TB, 1), lambda i: (i, 0)),
            _full((1, E)), _full((1, E)), _full((1, 1)),
        ],
        out_shape=[
            jax.ShapeDtypeStruct((N, DIM), jnp.float32),
            jax.ShapeDtypeStruct((N, 1), jnp.float32),
            jax.ShapeDtypeStruct((1, E), jnp.float32),
            jax.ShapeDtypeStruct((1, E), jnp.float32),
            jax.ShapeDtypeStruct((1, 1), jnp.float32),
        ],
    )(h2, r1(log_A), router_w, ew1, eb1.reshape(E, 1, HID), ew2,
      eb2.reshape(E, 1, DIM), sw1, sb1.reshape(S, 1, HID), sw2,
      sb2.reshape(S, 1, DIM), hw1, r1(hb1), hw2, hb2.reshape(1, 1),
      r1(n3_g), r1(n3_b))

    h3 = h3.reshape(B, N, DIM)
    halt = halt.reshape(B, N)
    loss = loss.reshape(())
    return h3, halt, halt, loss
